# baseline (speedup 1.0000x reference)
import jax
import jax.numpy as jnp
from jax import lax
from jax.experimental import pallas as pl
from jax.experimental.pallas import tpu as pltpu
from jax.experimental.pallas import tpu_sc as plsc

_EPS = 1e-05
_IOU_THR = 0.5
_NP = 5120
_NG = 512
_NEG = -jnp.inf
_NEGF = -1e30
_NSUB = 16
_NCORE = 2
_CHUNK = _NP // _NSUB
_HALF = _NG // _NCORE
_GPW = _HALF // _NSUB
_PH = 1


def _tc1_body(pred_ref, gt_ref, chosen_ref, cand_ref, cnt_ref):
    p = pred_ref[...]
    g = gt_ref[...]
    px1 = p[0:1, :]; py1 = p[1:2, :]; px2 = p[2:3, :]; py2 = p[3:4, :]
    plab = p[5:6, :]
    gx1 = g[:, 0:1]; gy1 = g[:, 1:2]; gx2 = g[:, 2:3]; gy2 = g[:, 3:4]
    glab = g[:, 4:5]
    area_p = (px2 - px1) * (py2 - py1)
    area_g = (gx2 - gx1) * (gy2 - gy1)
    w = jnp.maximum(jnp.minimum(gx2, px2) - jnp.maximum(gx1, px1), 0.0)
    h = jnp.maximum(jnp.minimum(gy2, py2) - jnp.maximum(gy1, py1), 0.0)
    inter = w * h
    iou = inter / (area_g + area_p - inter + 1e-12)
    rowid = jax.lax.broadcasted_iota(jnp.int32, (_NG, _NP), 0)
    mcls = glab == plab
    iou_m = jnp.where(mcls, iou, 0.0)
    maxv = jnp.max(iou_m, axis=0, keepdims=True)
    chosen = jnp.min(jnp.where(iou_m == maxv, rowid, _NG), axis=0, keepdims=True)
    cand = (plab >= 1.0) & (maxv > _IOU_THR)
    chosen_ref[...] = chosen
    cand_ref[...] = cand.astype(jnp.int32)
    lane16 = jax.lax.broadcasted_iota(jnp.int32, (1, 16), 1)
    cnt = jnp.zeros((1, 16), jnp.float32)
    for ci, c in enumerate((1.0, 2.0, 3.0)):
        num_gt = jnp.sum((glab == c).astype(jnp.float32))
        nmem = jnp.sum((plab == c).astype(jnp.float32))
        cnt = cnt + jnp.where(lane16 == ci, num_gt, 0.0)
        cnt = cnt + jnp.where(lane16 == ci + 3, nmem, 0.0)
    cnt_ref[...] = cnt


def _sc_body(chosen_hbm, cand_hbm, key_hbm, plab_hbm, glab_hbm,
             m1_hbm, widx_hbm, r_hbm,
             ch_v, cand_v, key_v, keyall_v, plaball_v,
             m1loc, widxloc, tmpm, tmpw, mfin, wfin, glab_v, r_v,
             m1_sh, widx_sh):
    s = lax.axis_index("s")
    c = lax.axis_index("c")
    lo = c * _HALF
    pbase = s * _CHUNK
    lane = lax.broadcasted_iota(jnp.int32, (16,), 0)

    pltpu.sync_copy(chosen_hbm.at[pl.ds(pbase, _CHUNK)],
                    ch_v.at[pl.ds(0, _CHUNK)])
    pltpu.sync_copy(cand_hbm.at[pl.ds(pbase, _CHUNK)],
                    cand_v.at[pl.ds(0, _CHUNK)])
    pltpu.sync_copy(key_hbm.at[pl.ds(pbase, _CHUNK)],
                    key_v.at[pl.ds(0, _CHUNK)])
    pltpu.sync_copy(key_hbm.at[:], keyall_v)
    pltpu.sync_copy(plab_hbm.at[:], plaball_v)

    def _init(j, _):
        m1loc[pl.ds(j * 16, 16)] = jnp.full((16,), _NEGF, jnp.float32)
        widxloc[pl.ds(j * 16, 16)] = jnp.full((16,), _NP, jnp.int32)
        return 0
    lax.fori_loop(0, (_HALF + 16) // 16, _init, 0)

    _phase1(ch_v, cand_v, key_v, m1loc, widxloc, lane, lo, pbase)
    _phase23(s, lo, m1loc, widxloc, m1_sh, widx_sh, tmpm, tmpw,
             mfin, wfin, glab_v, r_v, keyall_v, plaball_v, lane,
             m1_hbm, widx_hbm, r_hbm, glab_hbm)


def _phase1(ch_v, cand_v, key_v, m1loc, widxloc, lane, lo, pbase):
    def _scan(i, _):
        g = ch_v[pl.ds(i, 16)][0]
        ok = cand_v[pl.ds(i, 16)][0] == 1
        p = key_v[pl.ds(i, 16)][0]
        gl = g - lo
        inhalf = (gl >= 0) & (gl < _HALF)
        glc = jnp.clip(gl, 0, _HALF - 1)
        curm = m1loc[pl.ds(glc, 16)]
        curw = widxloc[pl.ds(glc, 16)]
        c0m = curm[0]
        c0w = curw[0]
        myidx = pbase + i
        take = ok & inhalf & ((p > c0m) | ((p == c0m) & (myidx < c0w)))
        newm = jnp.where(take, p, c0m)
        neww = jnp.where(take, myidx, c0w)
        m1loc[pl.ds(glc, 16)] = jnp.where(lane == 0, newm, curm)
        widxloc[pl.ds(glc, 16)] = jnp.where(lane == 0, neww, curw)
        return 0
    lax.fori_loop(0, _CHUNK, _scan, 0)


def _phase23(s, lo, m1loc, widxloc, m1_sh, widx_sh, tmpm, tmpw,
             mfin, wfin, glab_v, r_v, keyall_v, plaball_v, lane,
             m1_hbm, widx_hbm, r_hbm, glab_hbm):
    pltpu.sync_copy(m1loc.at[pl.ds(0, _HALF)],
                    m1_sh.at[pl.ds(s * _HALF, _HALF)])
    pltpu.sync_copy(widxloc.at[pl.ds(0, _HALF)],
                    widx_sh.at[pl.ds(s * _HALF, _HALF)])
    plsc.subcore_barrier()

    glo = s * _GPW
    def _merge(t, carry):
        macc, wacc = carry
        pltpu.sync_copy(m1_sh.at[pl.ds(t * _HALF + glo, _GPW)], tmpm)
        pltpu.sync_copy(widx_sh.at[pl.ds(t * _HALF + glo, _GPW)], tmpw)
        mv = tmpm[...]
        wv = tmpw[...]
        take = (mv > macc) | ((mv == macc) & (wv < wacc))
        return (jnp.where(take, mv, macc), jnp.where(take, wv, wacc))
    macc, wacc = lax.fori_loop(
        0, _NSUB, _merge,
        (jnp.full((_GPW,), _NEGF, jnp.float32), jnp.full((_GPW,), _NP, jnp.int32)))
    mfin[pl.ds(0, _GPW)] = macc
    wfin[pl.ds(0, _GPW)] = wacc
    pltpu.sync_copy(mfin.at[pl.ds(0, _GPW)], m1_hbm.at[pl.ds(lo + glo, _GPW)])
    pltpu.sync_copy(wfin.at[pl.ds(0, _GPW)], widx_hbm.at[pl.ds(lo + glo, _GPW)])

    pltpu.sync_copy(glab_hbm.at[pl.ds(lo + glo, _GPW)],
                    glab_v.at[pl.ds(0, _GPW)])

    def _per_gt(t, _):
        m1g = jnp.zeros((16,), jnp.float32) + mfin[pl.ds(t, 16)][0]
        wg = jnp.zeros((16,), jnp.int32) + wfin[pl.ds(t, 16)][0]
        glg = jnp.zeros((16,), jnp.float32) + glab_v[pl.ds(t, 16)][0]

        def _count(j, acc):
            k16 = keyall_v[pl.ds(j * 16, 16)]
            p16 = plaball_v[pl.ds(j * 16, 16)]
            idx16 = lane + j * 16
            beats = (p16 == glg) & ((k16 > m1g) | ((k16 == m1g) & (idx16 < wg)))
            return acc + jnp.where(beats, 1.0, 0.0)

        acc = lax.fori_loop(0, _NP // 16, _count,
                            jnp.zeros((16,), jnp.float32))
        r_v[pl.ds(t * 16, 16)] = acc
        return 0
    lax.fori_loop(0, _GPW, _per_gt, 0)
    pltpu.sync_copy(r_v.at[pl.ds(0, _GPW * 16)],
                    r_hbm.at[pl.ds((lo + glo) * 16, _GPW * 16)])


def _tc2_body(m1c_ref, m1r_ref, wc_ref, wr_ref, glc_ref, glr_ref, rc_ref,
              cnt_ref, tpts_ref, out_ref):
    m1 = m1c_ref[...]
    m1t = m1r_ref[...]
    wc = wc_ref[...]
    wt = wr_ref[...]
    gl = glc_ref[...]
    glt = glr_ref[...]
    r = jnp.sum(rc_ref[...], axis=1, keepdims=True)
    cnt = cnt_ref[...]
    tpts = tpts_ref[...]
    exists = m1 > -1e29
    existst = m1t > -1e29
    betterw = (existst & (glt == gl)) & (
        (m1t > m1) | ((m1t == m1) & (wt < wc)))
    k = 1.0 + jnp.sum(betterw.astype(jnp.float32), axis=1, keepdims=True)
    prec = k / (r + 1.0 + _EPS)
    total = jnp.float32(0.0)
    for ci, c in enumerate((1.0, 2.0, 3.0)):
        num_gt = cnt[0, ci]
        nmem = cnt[0, ci + 3]
        recall = k / (num_gt + _EPS)
        elig = (exists & (gl == c)) & (recall >= tpts)
        pmax = jnp.max(jnp.where(elig, prec, _NEG), axis=0, keepdims=True)
        any_e = jnp.max(elig.astype(jnp.float32), axis=0, keepdims=True) > 0
        ap = jnp.sum(jnp.where(any_e, pmax, 0.0)) / 11.0
        valid = (nmem > 0) & (num_gt > 0)
        total = total + jnp.where(valid, ap, 0.0)
    out_ref[...] = jnp.broadcast_to(total / 3.0, (1, 128))


_sc_mesh = plsc.VectorSubcoreMesh(core_axis_name="c", subcore_axis_name="s")

_sc_call = pl.kernel(
    _sc_body,
    out_type=[
        jax.ShapeDtypeStruct((_NG,), jnp.float32),
        jax.ShapeDtypeStruct((_NG,), jnp.int32),
        jax.ShapeDtypeStruct((_NG * 16,), jnp.float32),
    ],
    mesh=_sc_mesh,
    scratch_types=[
        pltpu.VMEM((_CHUNK + 16,), jnp.int32),
        pltpu.VMEM((_CHUNK + 16,), jnp.int32),
        pltpu.VMEM((_CHUNK + 16,), jnp.float32),
        pltpu.VMEM((_NP,), jnp.float32),
        pltpu.VMEM((_NP,), jnp.float32),
        pltpu.VMEM((_HALF + 16,), jnp.float32),
        pltpu.VMEM((_HALF + 16,), jnp.int32),
        pltpu.VMEM((_GPW,), jnp.float32),
        pltpu.VMEM((_GPW,), jnp.int32),
        pltpu.VMEM((_GPW + 16,), jnp.float32),
        pltpu.VMEM((_GPW + 16,), jnp.int32),
        pltpu.VMEM((_GPW + 16,), jnp.float32),
        pltpu.VMEM((_GPW * 16,), jnp.float32),
        pltpu.VMEM_SHARED((_NSUB * _HALF,), jnp.float32),
        pltpu.VMEM_SHARED((_NSUB * _HALF,), jnp.int32),
    ],
)


def kernel(pred_labels, class_probits, pred_boxes, gt_labels, gt_boxes):
    np0 = pred_boxes.shape[0]
    ng0 = gt_boxes.shape[0]
    pred = jnp.zeros((8, _NP), jnp.float32)
    pred = pred.at[0:4, :np0].set(pred_boxes.T.astype(jnp.float32))
    pred = pred.at[4, :np0].set(class_probits.astype(jnp.float32))
    pred = pred.at[5, :np0].set(pred_labels.astype(jnp.float32))
    pred = pred.at[5, np0:].set(-1.0)
    gt = jnp.zeros((_NG, 8), jnp.float32)
    gt = gt.at[:ng0, 0:4].set(gt_boxes.astype(jnp.float32))
    gt = gt.at[:ng0, 4].set(gt_labels.astype(jnp.float32))
    gt = gt.at[ng0:, 4].set(-2.0)
    tpts = jnp.full((1, 16), 2.0, jnp.float32)
    tpts = tpts.at[0, :11].set(jnp.arange(0.0, 1.1, 0.1, dtype=jnp.float32))

    chosen, cand, cnt = pl.pallas_call(
        _tc1_body,
        out_shape=[
            jax.ShapeDtypeStruct((1, _NP), jnp.int32),
            jax.ShapeDtypeStruct((1, _NP), jnp.int32),
            jax.ShapeDtypeStruct((1, 16), jnp.float32),
        ],
    )(pred, gt)

    key1 = pred[4, :]
    plab1 = pred[5, :]
    glab1 = gt[:, 4]
    m1, widx, r = _sc_call(
        chosen.reshape(_NP), cand.reshape(_NP), key1, plab1, glab1)

    out = pl.pallas_call(
        _tc2_body,
        out_shape=jax.ShapeDtypeStruct((1, 128), jnp.float32),
    )(
        m1.reshape(_NG, 1), m1.reshape(1, _NG),
        widx.astype(jnp.float32).reshape(_NG, 1),
        widx.astype(jnp.float32).reshape(1, _NG),
        glab1.reshape(_NG, 1), glab1.reshape(1, _NG),
        r.reshape(_NG, 16),
        cnt, tpts,
    )
    return out[0, 0]
